# baseline (speedup 1.0000x reference)
import functools

import jax
import jax.numpy as jnp
import numpy as np
from jax import lax
from jax.experimental import pallas as pl

_H = 128
_NR = 6
_NS = 7
_IE = 64
_OE = 256
_NB = 4
_CUTOFF = 5.0
_M = 16
_BMOL = 256
_MB = 8

_FREQS = np.pi * np.arange(1, _NR + 1, dtype=np.float32)
_EA = -28.0
_EB = 48.0
_EC = -21.0


def _silu(x):
    return x * jax.nn.sigmoid(x)


def _mmb(x, wref):
    return lax.dot_general(
        x.astype(jnp.bfloat16), wref[...],
        (((x.ndim - 1,), (0,)), ((), ())),
        preferred_element_type=jnp.float32)


def _is_bf16(name):
    if name in ("wer",):
        return True
    for tag in ("_kjw", "_jiw", "_down", "_up", "_lw",
                "_bf00w", "_bf01w", "_af00w", "_af01w", "_af10w", "_af11w",
                "_l0w", "_l1w", "_l2w"):
        if name.endswith(tag):
            return True
    return False


def _prep_weights(params):
    names, arrs = [], []

    def add(name, a):
        names.append(name)
        arrs.append(a)

    add("emb", params["emb"])
    add("wre", params["emb_lin_rbf"]["w"])
    add("bre", params["emb_lin_rbf"]["b"].reshape(1, -1))
    W = params["emb_lin"]["w"]
    add("wei", W[0:_H])
    add("wej", W[_H:2 * _H])
    add("wer", W[2 * _H:3 * _H])
    add("bemb", params["emb_lin"]["b"].reshape(1, -1))
    for b, ob in enumerate(params["out_blocks"]):
        add(f"o{b}_rbf", ob["lin_rbf"]["w"])
        add(f"o{b}_up", ob["lin_up"]["w"])
        for l, lin in enumerate(ob["lins"]):
            add(f"o{b}_l{l}w", lin["w"])
            add(f"o{b}_l{l}b", lin["b"].reshape(1, -1))
        add(f"o{b}_out", ob["lin_out"]["w"])
    for b, ib in enumerate(params["int_blocks"]):
        add(f"i{b}_rbfg", ib["lin_rbf1"]["w"] @ ib["lin_rbf2"]["w"])
        W12 = ib["lin_sbf1"]["w"] @ ib["lin_sbf2"]["w"]
        add(f"i{b}_sbfg",
            W12.reshape(_NS, _NR, _IE).transpose(1, 0, 2).reshape(_NR, _NS * _IE))
        add(f"i{b}_kjw", ib["lin_kj"]["w"])
        add(f"i{b}_kjb", ib["lin_kj"]["b"].reshape(1, -1))
        add(f"i{b}_jiw", ib["lin_ji"]["w"])
        add(f"i{b}_jib", ib["lin_ji"]["b"].reshape(1, -1))
        add(f"i{b}_down", ib["lin_down"]["w"])
        add(f"i{b}_up", ib["lin_up"]["w"])
        for r, pair in enumerate(ib["before"]):
            add(f"i{b}_bf{r}0w", pair[0]["w"])
            add(f"i{b}_bf{r}0b", pair[0]["b"].reshape(1, -1))
            add(f"i{b}_bf{r}1w", pair[1]["w"])
            add(f"i{b}_bf{r}1b", pair[1]["b"].reshape(1, -1))
        add(f"i{b}_lw", ib["lin"]["w"])
        add(f"i{b}_lb", ib["lin"]["b"].reshape(1, -1))
        for r, pair in enumerate(ib["after"]):
            add(f"i{b}_af{r}0w", pair[0]["w"])
            add(f"i{b}_af{r}0b", pair[0]["b"].reshape(1, -1))
            add(f"i{b}_af{r}1w", pair[1]["w"])
            add(f"i{b}_af{r}1b", pair[1]["b"].reshape(1, -1))
    arrs = [a.astype(jnp.bfloat16) if _is_bf16(n) else a
            for n, a in zip(names, arrs)]
    return names, arrs


def _out_block(w, b, R2, h4):
    g = (R2 @ w[f"o{b}_rbf"][...]).reshape(_MB, _M, _M, _H) * h4
    t = jnp.sum(g, axis=1)
    t2 = _mmb(t.reshape(_MB * _M, _H), w[f"o{b}_up"])
    for l in range(3):
        t2 = _silu(_mmb(t2, w[f"o{b}_l{l}w"]) + w[f"o{b}_l{l}b"][...])
    ms = jnp.sum(t2.reshape(_MB, _M, _OE), axis=1)
    return (ms @ w[f"o{b}_out"][...]).reshape(_MB)


def _int_block(w, b, R2, cbf, h4):
    hflat = h4.reshape(_MB * _M * _M, _H)
    x_ji = _silu(_mmb(hflat, w[f"i{b}_jiw"]) + w[f"i{b}_jib"][...])
    x_kj = _silu(_mmb(hflat, w[f"i{b}_kjw"]) + w[f"i{b}_kjb"][...])
    x_kj = x_kj * (R2 @ w[f"i{b}_rbfg"][...])
    mfe = _silu(_mmb(x_kj, w[f"i{b}_down"]))
    mT = mfe.reshape(_MB, _M, _M, _IE).swapaxes(1, 2)
    mT = mT.reshape(_MB * _M, _M, _IE)
    BB = (R2 @ w[f"i{b}_sbfg"][...]).reshape(_MB * _M, _M, _NS * _IE)
    C = jnp.concatenate(
        [mT * BB[:, :, s * _IE:(s + 1) * _IE] for s in range(_NS)],
        axis=1)
    agg = lax.dot_general(cbf, C, (((2,), (1,)), ((0,), (0,))))
    x_up = _silu(_mmb(agg.reshape(_MB * _M * _M, _IE), w[f"i{b}_up"]))
    hn = x_ji + x_up
    hn = hn + _silu(_mmb(_silu(_mmb(hn, w[f"i{b}_bf00w"])
                               + w[f"i{b}_bf00b"][...]),
                         w[f"i{b}_bf01w"]) + w[f"i{b}_bf01b"][...])
    hn = _silu(_mmb(hn, w[f"i{b}_lw"]) + w[f"i{b}_lb"][...]) + hflat
    for r in range(2):
        hn = hn + _silu(
            _mmb(_silu(_mmb(hn, w[f"i{b}_af{r}0w"]) + w[f"i{b}_af{r}0b"][...]),
                 w[f"i{b}_af{r}1w"]) + w[f"i{b}_af{r}1b"][...])
    return hn.reshape(_MB, _M, _M, _H)


def _body(wnames, *refs):
    px_r, py_r, pz_r, zoh_r = refs[0:4]
    out_r = refs[-1]
    w = dict(zip(wnames, refs[4:-1]))

    px = px_r[...]
    py = py_r[...]
    pz = pz_r[...]
    dx = px[:, None, :] - px[:, :, None]
    dy = py[:, None, :] - py[:, :, None]
    dz = pz[:, None, :] - pz[:, :, None]
    ia = lax.broadcasted_iota(jnp.int32, (_M, _M), 0)
    ib = lax.broadcasted_iota(jnp.int32, (_M, _M), 1)
    eye = (ia == ib).astype(jnp.float32)
    dist2 = dx * dx + dy * dy + dz * dz + 100.0 * eye[None]
    rd = lax.rsqrt(dist2)
    dist = dist2 * rd
    x01 = dist * (1.0 / _CUTOFF)
    env = 1.0 / x01 + (x01 ** 5) * (_EA + x01 * (_EB + x01 * _EC))
    env = jnp.where(x01 < 1.0, env, 0.0)
    ang = x01 * np.float32(np.pi)
    s1 = jnp.sin(ang)
    c2 = 2.0 * jnp.cos(ang)
    sins = [s1]
    sprev = jnp.zeros_like(s1)
    for _ in range(1, _NR):
        snew = c2 * sins[-1] - sprev
        sprev = sins[-1]
        sins.append(snew)
    rbf4 = env[..., None] * jnp.concatenate(
        [s[..., None] for s in sins], axis=-1)
    R2 = rbf4.reshape(_MB * _M * _M, _NR)

    ux = dx * rd
    uy = dy * rd
    uz = dz * rd
    cth = (ux[:, :, :, None] * ux[:, :, None, :]
           + uy[:, :, :, None] * uy[:, :, None, :]
           + uz[:, :, :, None] * uz[:, :, None, :])
    kimask = (ia != ib).astype(jnp.float32)[None, None]
    parts = []
    tprev = jnp.ones_like(cth)
    tcur = cth
    for s in range(_NS):
        if s == 0:
            t = tprev
        elif s == 1:
            t = tcur
        else:
            t = 2.0 * cth * tcur - tprev
            tprev, tcur = tcur, t
        parts.append(t * kimask)
    cbf = jnp.concatenate(parts, axis=-1)
    cbf = cbf.reshape(_MB * _M, _M, _NS * _M)

    xz = zoh_r[...].reshape(_MB * _M, 95) @ w["emb"][...]
    rbf_e = _silu(R2 @ w["wre"][...] + w["bre"][...])
    xzi = (xz @ w["wei"][...]).reshape(_MB, _M, _H)
    xzj = (xz @ w["wej"][...]).reshape(_MB, _M, _H)
    h4 = _silu(xzi[:, None, :, :] + xzj[:, :, None, :]
               + _mmb(rbf_e, w["wer"]).reshape(_MB, _M, _M, _H)
               + w["bemb"][...])

    P = _out_block(w, 0, R2, h4)
    for b in range(_NB):
        h4 = _int_block(w, b, R2, cbf, h4)
        P = P + _out_block(w, b + 1, R2, h4)
    out_r[...] = P.reshape(_MB, 1)


def kernel(z, pos, batch, params):
    del batch
    wnames, warrs = _prep_weights(params)
    posr = pos.reshape(_BMOL, _M, 3)
    px = posr[:, :, 0]
    py = posr[:, :, 1]
    pz = posr[:, :, 2]
    zoh = jax.nn.one_hot(z.reshape(_BMOL, _M), 95, dtype=jnp.float32)

    in_specs = [
        pl.BlockSpec((_MB, _M), lambda i: (i, 0)),
        pl.BlockSpec((_MB, _M), lambda i: (i, 0)),
        pl.BlockSpec((_MB, _M), lambda i: (i, 0)),
        pl.BlockSpec((_MB, _M, 95), lambda i: (i, 0, 0)),
    ]
    for a in warrs:
        nd = a.ndim
        in_specs.append(
            pl.BlockSpec(a.shape, lambda i, _nd=nd: (0,) * _nd))

    fn = pl.pallas_call(
        functools.partial(_body, wnames),
        grid=(_BMOL // _MB,),
        in_specs=in_specs,
        out_specs=pl.BlockSpec((_MB, 1), lambda i: (i, 0)),
        out_shape=jax.ShapeDtypeStruct((_BMOL, 1), jnp.float32),
    )
    return fn(px, py, pz, zoh, *warrs)


# speedup vs baseline: 1.0529x; 1.0529x over previous
import functools

import jax
import jax.numpy as jnp
import numpy as np
from jax import lax
from jax.experimental import pallas as pl
from jax.experimental.pallas import tpu as pltpu

_H = 128
_NR = 6
_NS = 7
_IE = 64
_OE = 256
_NB = 4
_CUTOFF = 5.0
_M = 16
_BMOL = 256
_MB = 8

_FREQS = np.pi * np.arange(1, _NR + 1, dtype=np.float32)
_EA = -28.0
_EB = 48.0
_EC = -21.0


def _silu(x):
    return x * jax.nn.sigmoid(x)


def _mmb(x, wref):
    return x @ wref[...]


def _is_bf16(name):
    if name in ("wer",):
        return True
    for tag in ("_kjw", "_jiw", "_down", "_up", "_lw",
                "_bf00w", "_bf01w", "_af00w", "_af01w", "_af10w", "_af11w",
                "_l0w", "_l1w", "_l2w"):
        if name.endswith(tag):
            return True
    return False


def _prep_weights(params):
    names, arrs = [], []

    def add(name, a):
        names.append(name)
        arrs.append(a)

    add("emb", params["emb"])
    add("wre", params["emb_lin_rbf"]["w"])
    add("bre", params["emb_lin_rbf"]["b"].reshape(1, -1))
    W = params["emb_lin"]["w"]
    add("wei", W[0:_H])
    add("wej", W[_H:2 * _H])
    add("wer", W[2 * _H:3 * _H])
    add("bemb", params["emb_lin"]["b"].reshape(1, -1))
    for b, ob in enumerate(params["out_blocks"]):
        add(f"o{b}_rbf", ob["lin_rbf"]["w"])
        add(f"o{b}_up", ob["lin_up"]["w"])
        for l, lin in enumerate(ob["lins"]):
            add(f"o{b}_l{l}w", lin["w"])
            add(f"o{b}_l{l}b", lin["b"].reshape(1, -1))
        add(f"o{b}_out", ob["lin_out"]["w"])
    for b, ib in enumerate(params["int_blocks"]):
        add(f"i{b}_rbfg", ib["lin_rbf1"]["w"] @ ib["lin_rbf2"]["w"])
        W12 = ib["lin_sbf1"]["w"] @ ib["lin_sbf2"]["w"]
        add(f"i{b}_sbfg",
            W12.reshape(_NS, _NR, _IE).transpose(1, 0, 2).reshape(_NR, _NS * _IE))
        add(f"i{b}_kjw", ib["lin_kj"]["w"])
        add(f"i{b}_kjb", ib["lin_kj"]["b"].reshape(1, -1))
        add(f"i{b}_jiw", ib["lin_ji"]["w"])
        add(f"i{b}_jib", ib["lin_ji"]["b"].reshape(1, -1))
        add(f"i{b}_down", ib["lin_down"]["w"])
        add(f"i{b}_up", ib["lin_up"]["w"])
        for r, pair in enumerate(ib["before"]):
            add(f"i{b}_bf{r}0w", pair[0]["w"])
            add(f"i{b}_bf{r}0b", pair[0]["b"].reshape(1, -1))
            add(f"i{b}_bf{r}1w", pair[1]["w"])
            add(f"i{b}_bf{r}1b", pair[1]["b"].reshape(1, -1))
        add(f"i{b}_lw", ib["lin"]["w"])
        add(f"i{b}_lb", ib["lin"]["b"].reshape(1, -1))
        for r, pair in enumerate(ib["after"]):
            add(f"i{b}_af{r}0w", pair[0]["w"])
            add(f"i{b}_af{r}0b", pair[0]["b"].reshape(1, -1))
            add(f"i{b}_af{r}1w", pair[1]["w"])
            add(f"i{b}_af{r}1b", pair[1]["b"].reshape(1, -1))
    return names, arrs


def _out_block(w, b, R2, h4):
    g = (R2 @ w[f"o{b}_rbf"][...]).reshape(_MB, _M, _M, _H) * h4
    t = jnp.sum(g, axis=1)
    t2 = _mmb(t.reshape(_MB * _M, _H), w[f"o{b}_up"])
    for l in range(3):
        t2 = _silu(_mmb(t2, w[f"o{b}_l{l}w"]) + w[f"o{b}_l{l}b"][...])
    ms = jnp.sum(t2.reshape(_MB, _M, _OE), axis=1)
    return (ms @ w[f"o{b}_out"][...]).reshape(_MB)


def _int_block(w, b, R2, cbf, h4):
    hflat = h4.reshape(_MB * _M * _M, _H)
    x_ji = _silu(_mmb(hflat, w[f"i{b}_jiw"]) + w[f"i{b}_jib"][...])
    x_kj = _silu(_mmb(hflat, w[f"i{b}_kjw"]) + w[f"i{b}_kjb"][...])
    x_kj = x_kj * (R2 @ w[f"i{b}_rbfg"][...])
    mfe = _silu(_mmb(x_kj, w[f"i{b}_down"]))
    mT = mfe.reshape(_MB, _M, _M, _IE).swapaxes(1, 2)
    mT = mT.reshape(_MB * _M, _M, _IE)
    BB = (R2 @ w[f"i{b}_sbfg"][...]).reshape(_MB * _M, _M, _NS * _IE)
    C = jnp.concatenate(
        [mT * BB[:, :, s * _IE:(s + 1) * _IE] for s in range(_NS)],
        axis=1)
    agg = lax.dot_general(cbf, C, (((2,), (1,)), ((0,), (0,))))
    x_up = _silu(_mmb(agg.reshape(_MB * _M * _M, _IE), w[f"i{b}_up"]))
    hn = x_ji + x_up
    hn = hn + _silu(_mmb(_silu(_mmb(hn, w[f"i{b}_bf00w"])
                               + w[f"i{b}_bf00b"][...]),
                         w[f"i{b}_bf01w"]) + w[f"i{b}_bf01b"][...])
    hn = _silu(_mmb(hn, w[f"i{b}_lw"]) + w[f"i{b}_lb"][...]) + hflat
    for r in range(2):
        hn = hn + _silu(
            _mmb(_silu(_mmb(hn, w[f"i{b}_af{r}0w"]) + w[f"i{b}_af{r}0b"][...]),
                 w[f"i{b}_af{r}1w"]) + w[f"i{b}_af{r}1b"][...])
    return hn.reshape(_MB, _M, _M, _H)


def _body(wnames, *refs):
    px_r, py_r, pz_r, zoh_r = refs[0:4]
    out_r = refs[-1]
    w = dict(zip(wnames, refs[4:-1]))

    px = px_r[...]
    py = py_r[...]
    pz = pz_r[...]
    dx = px[:, None, :] - px[:, :, None]
    dy = py[:, None, :] - py[:, :, None]
    dz = pz[:, None, :] - pz[:, :, None]
    ia = lax.broadcasted_iota(jnp.int32, (_M, _M), 0)
    ib = lax.broadcasted_iota(jnp.int32, (_M, _M), 1)
    eye = (ia == ib).astype(jnp.float32)
    dist2 = dx * dx + dy * dy + dz * dz + 100.0 * eye[None]
    rd = lax.rsqrt(dist2)
    dist = dist2 * rd
    x01 = dist * (1.0 / _CUTOFF)
    env = 1.0 / x01 + (x01 ** 5) * (_EA + x01 * (_EB + x01 * _EC))
    env = jnp.where(x01 < 1.0, env, 0.0)
    ang = x01 * np.float32(np.pi)
    s1 = jnp.sin(ang)
    c2 = 2.0 * jnp.cos(ang)
    sins = [s1]
    sprev = jnp.zeros_like(s1)
    for _ in range(1, _NR):
        snew = c2 * sins[-1] - sprev
        sprev = sins[-1]
        sins.append(snew)
    rbf4 = env[..., None] * jnp.concatenate(
        [s[..., None] for s in sins], axis=-1)
    R2 = rbf4.reshape(_MB * _M * _M, _NR)

    ux = dx * rd
    uy = dy * rd
    uz = dz * rd
    cth = (ux[:, :, :, None] * ux[:, :, None, :]
           + uy[:, :, :, None] * uy[:, :, None, :]
           + uz[:, :, :, None] * uz[:, :, None, :])
    kimask = (ia != ib).astype(jnp.float32)[None, None]
    parts = []
    tprev = jnp.ones_like(cth)
    tcur = cth
    for s in range(_NS):
        if s == 0:
            t = tprev
        elif s == 1:
            t = tcur
        else:
            t = 2.0 * cth * tcur - tprev
            tprev, tcur = tcur, t
        parts.append(t * kimask)
    cbf = jnp.concatenate(parts, axis=-1)
    cbf = cbf.reshape(_MB * _M, _M, _NS * _M)

    xz = zoh_r[...].reshape(_MB * _M, 95) @ w["emb"][...]
    rbf_e = _silu(R2 @ w["wre"][...] + w["bre"][...])
    xzi = (xz @ w["wei"][...]).reshape(_MB, _M, _H)
    xzj = (xz @ w["wej"][...]).reshape(_MB, _M, _H)
    h4 = _silu(xzi[:, None, :, :] + xzj[:, :, None, :]
               + _mmb(rbf_e, w["wer"]).reshape(_MB, _M, _M, _H)
               + w["bemb"][...])

    P = _out_block(w, 0, R2, h4)
    for b in range(_NB):
        h4 = _int_block(w, b, R2, cbf, h4)
        P = P + _out_block(w, b + 1, R2, h4)
    out_r[...] = P.reshape(_MB, 1)


def kernel(z, pos, batch, params):
    del batch
    wnames, warrs = _prep_weights(params)
    posr = pos.reshape(_BMOL, _M, 3)
    px = posr[:, :, 0]
    py = posr[:, :, 1]
    pz = posr[:, :, 2]
    zoh = jax.nn.one_hot(z.reshape(_BMOL, _M), 95, dtype=jnp.float32)

    in_specs = [
        pl.BlockSpec((_MB, _M), lambda i: (i, 0)),
        pl.BlockSpec((_MB, _M), lambda i: (i, 0)),
        pl.BlockSpec((_MB, _M), lambda i: (i, 0)),
        pl.BlockSpec((_MB, _M, 95), lambda i: (i, 0, 0)),
    ]
    for a in warrs:
        nd = a.ndim
        in_specs.append(
            pl.BlockSpec(a.shape, lambda i, _nd=nd: (0,) * _nd))

    fn = pl.pallas_call(
        functools.partial(_body, wnames),
        grid=(_BMOL // _MB,),
        in_specs=in_specs,
        out_specs=pl.BlockSpec((_MB, 1), lambda i: (i, 0)),
        out_shape=jax.ShapeDtypeStruct((_BMOL, 1), jnp.float32),
        compiler_params=pltpu.CompilerParams(
            dimension_semantics=("parallel",)),
    )
    return fn(px, py, pz, zoh, *warrs)


# speedup vs baseline: 1.1648x; 1.1063x over previous
import functools

import jax
import jax.numpy as jnp
import numpy as np
from jax import lax
from jax.experimental import pallas as pl
from jax.experimental.pallas import tpu as pltpu

_H = 128
_NR = 6
_NS = 7
_IE = 64
_OE = 256
_NB = 4
_CUTOFF = 5.0
_M = 16
_BMOL = 256
_MB = 8

_FREQS = np.pi * np.arange(1, _NR + 1, dtype=np.float32)
_EA = -28.0
_EB = 48.0
_EC = -21.0


def _silu(x):
    h = x * 0.5
    return h * jnp.tanh(h) + h


def _mmb(x, wref):
    return x @ wref[...]


def _is_bf16(name):
    if name in ("wer",):
        return True
    for tag in ("_kjw", "_jiw", "_down", "_up", "_lw",
                "_bf00w", "_bf01w", "_af00w", "_af01w", "_af10w", "_af11w",
                "_l0w", "_l1w", "_l2w"):
        if name.endswith(tag):
            return True
    return False


def _prep_weights(params):
    names, arrs = [], []

    def add(name, a):
        names.append(name)
        arrs.append(a)

    add("emb", params["emb"])
    add("wre", params["emb_lin_rbf"]["w"])
    add("bre", params["emb_lin_rbf"]["b"].reshape(1, -1))
    W = params["emb_lin"]["w"]
    add("wei", W[0:_H])
    add("wej", W[_H:2 * _H])
    add("wer", W[2 * _H:3 * _H])
    add("bemb", params["emb_lin"]["b"].reshape(1, -1))
    for b, ob in enumerate(params["out_blocks"]):
        add(f"o{b}_rbf", ob["lin_rbf"]["w"])
        add(f"o{b}_up", ob["lin_up"]["w"])
        for l, lin in enumerate(ob["lins"]):
            add(f"o{b}_l{l}w", lin["w"])
            add(f"o{b}_l{l}b", lin["b"].reshape(1, -1))
        add(f"o{b}_out", ob["lin_out"]["w"])
    for b, ib in enumerate(params["int_blocks"]):
        add(f"i{b}_rbfg", ib["lin_rbf1"]["w"] @ ib["lin_rbf2"]["w"])
        W12 = ib["lin_sbf1"]["w"] @ ib["lin_sbf2"]["w"]
        add(f"i{b}_sbfg",
            W12.reshape(_NS, _NR, _IE).transpose(1, 0, 2).reshape(_NR, _NS * _IE))
        add(f"i{b}_kjw", ib["lin_kj"]["w"])
        add(f"i{b}_kjb", ib["lin_kj"]["b"].reshape(1, -1))
        add(f"i{b}_jiw", ib["lin_ji"]["w"])
        add(f"i{b}_jib", ib["lin_ji"]["b"].reshape(1, -1))
        add(f"i{b}_down", ib["lin_down"]["w"])
        add(f"i{b}_up", ib["lin_up"]["w"])
        for r, pair in enumerate(ib["before"]):
            add(f"i{b}_bf{r}0w", pair[0]["w"])
            add(f"i{b}_bf{r}0b", pair[0]["b"].reshape(1, -1))
            add(f"i{b}_bf{r}1w", pair[1]["w"])
            add(f"i{b}_bf{r}1b", pair[1]["b"].reshape(1, -1))
        add(f"i{b}_lw", ib["lin"]["w"])
        add(f"i{b}_lb", ib["lin"]["b"].reshape(1, -1))
        for r, pair in enumerate(ib["after"]):
            add(f"i{b}_af{r}0w", pair[0]["w"])
            add(f"i{b}_af{r}0b", pair[0]["b"].reshape(1, -1))
            add(f"i{b}_af{r}1w", pair[1]["w"])
            add(f"i{b}_af{r}1b", pair[1]["b"].reshape(1, -1))
    return names, arrs


def _out_block(w, b, R2, h4):
    g = (R2 @ w[f"o{b}_rbf"][...]).reshape(_MB, _M, _M, _H) * h4
    t = jnp.sum(g, axis=1)
    t2 = _mmb(t.reshape(_MB * _M, _H), w[f"o{b}_up"])
    for l in range(3):
        t2 = _silu(_mmb(t2, w[f"o{b}_l{l}w"]) + w[f"o{b}_l{l}b"][...])
    ms = jnp.sum(t2.reshape(_MB, _M, _OE), axis=1)
    return (ms @ w[f"o{b}_out"][...]).reshape(_MB)


def _int_block(w, b, R2, cbf, h4):
    hflat = h4.reshape(_MB * _M * _M, _H)
    x_ji = _silu(_mmb(hflat, w[f"i{b}_jiw"]) + w[f"i{b}_jib"][...])
    x_kj = _silu(_mmb(hflat, w[f"i{b}_kjw"]) + w[f"i{b}_kjb"][...])
    x_kj = x_kj * (R2 @ w[f"i{b}_rbfg"][...])
    mfe = _silu(_mmb(x_kj, w[f"i{b}_down"]))
    mT = mfe.reshape(_MB, _M, _M, _IE).swapaxes(1, 2)
    mT = mT.reshape(_MB * _M, _M, _IE)
    BB = (R2 @ w[f"i{b}_sbfg"][...]).reshape(_MB * _M, _M, _NS * _IE)
    C = jnp.concatenate(
        [mT * BB[:, :, s * _IE:(s + 1) * _IE] for s in range(_NS)],
        axis=1)
    agg = lax.dot_general(cbf, C, (((2,), (1,)), ((0,), (0,))))
    x_up = _silu(_mmb(agg.reshape(_MB * _M * _M, _IE), w[f"i{b}_up"]))
    hn = x_ji + x_up
    hn = hn + _silu(_mmb(_silu(_mmb(hn, w[f"i{b}_bf00w"])
                               + w[f"i{b}_bf00b"][...]),
                         w[f"i{b}_bf01w"]) + w[f"i{b}_bf01b"][...])
    hn = _silu(_mmb(hn, w[f"i{b}_lw"]) + w[f"i{b}_lb"][...]) + hflat
    for r in range(2):
        hn = hn + _silu(
            _mmb(_silu(_mmb(hn, w[f"i{b}_af{r}0w"]) + w[f"i{b}_af{r}0b"][...]),
                 w[f"i{b}_af{r}1w"]) + w[f"i{b}_af{r}1b"][...])
    return hn.reshape(_MB, _M, _M, _H)


def _body(wnames, *refs):
    px_r, py_r, pz_r, zoh_r = refs[0:4]
    out_r = refs[-1]
    w = dict(zip(wnames, refs[4:-1]))

    px = px_r[...]
    py = py_r[...]
    pz = pz_r[...]
    dx = px[:, None, :] - px[:, :, None]
    dy = py[:, None, :] - py[:, :, None]
    dz = pz[:, None, :] - pz[:, :, None]
    ia = lax.broadcasted_iota(jnp.int32, (_M, _M), 0)
    ib = lax.broadcasted_iota(jnp.int32, (_M, _M), 1)
    eye = (ia == ib).astype(jnp.float32)
    dist2 = dx * dx + dy * dy + dz * dz + 100.0 * eye[None]
    rd = lax.rsqrt(dist2)
    dist = dist2 * rd
    x01 = dist * (1.0 / _CUTOFF)
    env = 1.0 / x01 + (x01 ** 5) * (_EA + x01 * (_EB + x01 * _EC))
    env = jnp.where(x01 < 1.0, env, 0.0)
    ang = x01 * np.float32(np.pi)
    s1 = jnp.sin(ang)
    c2 = 2.0 * jnp.cos(ang)
    sins = [s1]
    sprev = jnp.zeros_like(s1)
    for _ in range(1, _NR):
        snew = c2 * sins[-1] - sprev
        sprev = sins[-1]
        sins.append(snew)
    rbf4 = env[..., None] * jnp.concatenate(
        [s[..., None] for s in sins], axis=-1)
    R2 = rbf4.reshape(_MB * _M * _M, _NR)

    ux = dx * rd
    uy = dy * rd
    uz = dz * rd
    cth = (ux[:, :, :, None] * ux[:, :, None, :]
           + uy[:, :, :, None] * uy[:, :, None, :]
           + uz[:, :, :, None] * uz[:, :, None, :])
    kimask = (ia != ib).astype(jnp.float32)[None, None]
    parts = []
    tprev = jnp.ones_like(cth)
    tcur = cth
    for s in range(_NS):
        if s == 0:
            t = tprev
        elif s == 1:
            t = tcur
        else:
            t = 2.0 * cth * tcur - tprev
            tprev, tcur = tcur, t
        parts.append(t * kimask)
    cbf = jnp.concatenate(parts, axis=-1)
    cbf = cbf.reshape(_MB * _M, _M, _NS * _M)

    xz = zoh_r[...].reshape(_MB * _M, 95) @ w["emb"][...]
    rbf_e = _silu(R2 @ w["wre"][...] + w["bre"][...])
    xzi = (xz @ w["wei"][...]).reshape(_MB, _M, _H)
    xzj = (xz @ w["wej"][...]).reshape(_MB, _M, _H)
    h4 = _silu(xzi[:, None, :, :] + xzj[:, :, None, :]
               + _mmb(rbf_e, w["wer"]).reshape(_MB, _M, _M, _H)
               + w["bemb"][...])

    P = _out_block(w, 0, R2, h4)
    for b in range(_NB):
        h4 = _int_block(w, b, R2, cbf, h4)
        P = P + _out_block(w, b + 1, R2, h4)
    out_r[...] = P.reshape(_MB, 1)


def kernel(z, pos, batch, params):
    del batch
    wnames, warrs = _prep_weights(params)
    posr = pos.reshape(_BMOL, _M, 3)
    px = posr[:, :, 0]
    py = posr[:, :, 1]
    pz = posr[:, :, 2]
    zoh = jax.nn.one_hot(z.reshape(_BMOL, _M), 95, dtype=jnp.float32)

    in_specs = [
        pl.BlockSpec((_MB, _M), lambda i: (i, 0)),
        pl.BlockSpec((_MB, _M), lambda i: (i, 0)),
        pl.BlockSpec((_MB, _M), lambda i: (i, 0)),
        pl.BlockSpec((_MB, _M, 95), lambda i: (i, 0, 0)),
    ]
    for a in warrs:
        nd = a.ndim
        in_specs.append(
            pl.BlockSpec(a.shape, lambda i, _nd=nd: (0,) * _nd))

    fn = pl.pallas_call(
        functools.partial(_body, wnames),
        grid=(_BMOL // _MB,),
        in_specs=in_specs,
        out_specs=pl.BlockSpec((_MB, 1), lambda i: (i, 0)),
        out_shape=jax.ShapeDtypeStruct((_BMOL, 1), jnp.float32),
        compiler_params=pltpu.CompilerParams(
            dimension_semantics=("parallel",)),
    )
    return fn(px, py, pz, zoh, *warrs)
